# baseline (speedup 1.0000x reference)
import functools

import jax
import jax.numpy as jnp
from jax import lax
from jax.experimental import pallas as pl
from jax.experimental.pallas import tpu as pltpu
from jax.experimental.pallas import tpu_sc as plsc

B = 16384
D = 64
F = 16
H1 = 128
H2 = 64
NUSERS = 1000000
NMOVIES = 100000

NC = 2
NS = 16
NW = NC * NS
BPW = B // NW
CHUNK = 128
L = 16

_sc_mesh = plsc.VectorSubcoreMesh(core_axis_name="c", subcore_axis_name="s")

CB = 1024


def _reformat_body(tT_ref, o_ref):
    x = tT_ref[...]
    y = x.T.reshape(CB // 2, 2, D)
    o_ref[...] = jnp.concatenate([y[:, 0, :], y[:, 1, :]], axis=1)


def _reformat_tc(tT, n):
    grid = (pl.cdiv(n, CB),)
    return pl.pallas_call(
        _reformat_body,
        grid=grid,
        in_specs=[pl.BlockSpec((D, CB), lambda j: (0, j))],
        out_specs=pl.BlockSpec((CB // 2, 2 * D), lambda j: (j, 0)),
        out_shape=jax.ShapeDtypeStruct((n // 2, 2 * D), jnp.float32),
    )(tT)


def _gather_one(ids_hbm, pair_hbm, out_hbm, base, idx_v, tidx_v, rows_v, sem):
    pltpu.sync_copy(ids_hbm.at[pl.ds(base, BPW)], idx_v)

    def _shift(i, carry):
        sl = pl.ds(i * L, L)
        tidx_v[sl] = lax.shift_right_logical(idx_v[sl], 1)
        return carry

    lax.fori_loop(0, BPW // L, _shift, 0, unroll=4)

    copies = []
    for j in range(BPW // CHUNK):
        sl = pl.ds(j * CHUNK, CHUNK)
        copies.append(
            pltpu.async_copy(pair_hbm.at[tidx_v.at[sl]], rows_v.at[sl], sem))
    for cp in copies:
        cp.wait()
    pltpu.sync_copy(rows_v, out_hbm.at[pl.ds(base, BPW)])


@functools.partial(
    pl.kernel,
    mesh=_sc_mesh,
    out_type=[
        jax.ShapeDtypeStruct((B, 2 * D), jnp.float32),
        jax.ShapeDtypeStruct((B, 2 * D), jnp.float32),
    ],
    scratch_types=[
        pltpu.VMEM((BPW,), jnp.int32),
        pltpu.VMEM((BPW,), jnp.int32),
        pltpu.VMEM((BPW, 2 * D), jnp.float32),
        pltpu.SemaphoreType.DMA,
    ],
)
def _gather_sc(uids_hbm, mids_hbm, upair_hbm, mpair_hbm, uout_hbm, mout_hbm,
               idx_v, tidx_v, rows_v, sem):
    wid = lax.axis_index("s") * NC + lax.axis_index("c")
    base = wid * BPW
    _gather_one(uids_hbm, upair_hbm, uout_hbm, base, idx_v, tidx_v, rows_v,
                sem)
    _gather_one(mids_hbm, mpair_hbm, mout_hbm, base, idx_v, tidx_v, rows_v,
                sem)


BB = 512


def _mlp_body(u2_ref, m2_ref, uid_ref, mid_ref, f_ref, w1_ref, b1_ref,
              w2_ref, b2_ref, w3t_ref, b3_ref, o_ref):
    u2 = u2_ref[...]
    m2 = m2_ref[...]
    up = (uid_ref[...] & 1)[:, None] == 1
    mp = (mid_ref[...] & 1)[:, None] == 1
    u = jnp.where(up, u2[:, D:2 * D], u2[:, 0:D])
    m = jnp.where(mp, m2[:, D:2 * D], m2[:, 0:D])
    w1 = w1_ref[...]
    h1 = (
        jnp.dot(u, w1[0:D, :], preferred_element_type=jnp.float32)
        + jnp.dot(m, w1[D:2 * D, :], preferred_element_type=jnp.float32)
        + jnp.dot(f_ref[...], w1[2 * D:2 * D + F, :],
                  preferred_element_type=jnp.float32)
        + b1_ref[...]
    )
    h1 = jnp.maximum(h1, 0.0)
    h2 = jnp.maximum(
        jnp.dot(h1, w2_ref[...], preferred_element_type=jnp.float32)
        + b2_ref[...], 0.0)
    z = jnp.sum(h2 * w3t_ref[...], axis=1, keepdims=True) + b3_ref[...]
    o_ref[...] = jax.nn.sigmoid(z)


@jax.jit
def _mlp_tc(u2, m2, uids, mids, f, W1, b1, W2, b2, W3, b3):
    grid = (B // BB,)
    return pl.pallas_call(
        _mlp_body,
        grid=grid,
        in_specs=[
            pl.BlockSpec((BB, 2 * D), lambda i: (i, 0)),
            pl.BlockSpec((BB, 2 * D), lambda i: (i, 0)),
            pl.BlockSpec((BB,), lambda i: (i,)),
            pl.BlockSpec((BB,), lambda i: (i,)),
            pl.BlockSpec((BB, F), lambda i: (i, 0)),
            pl.BlockSpec((2 * D + F, H1), lambda i: (0, 0)),
            pl.BlockSpec((1, H1), lambda i: (0, 0)),
            pl.BlockSpec((H1, H2), lambda i: (0, 0)),
            pl.BlockSpec((1, H2), lambda i: (0, 0)),
            pl.BlockSpec((1, H2), lambda i: (0, 0)),
            pl.BlockSpec((1, 1), lambda i: (0, 0)),
        ],
        out_specs=pl.BlockSpec((BB, 1), lambda i: (i, 0)),
        out_shape=jax.ShapeDtypeStruct((B, 1), jnp.float32),
    )(u2, m2, uids, mids, f, W1, b1.reshape(1, H1), W2, b2.reshape(1, H2),
      W3.reshape(1, H2), b3.reshape(1, 1))


def kernel(user_ids, movie_ids, movie_features, user_table, movie_table,
           W1, b1, W2, b2, W3, b3):
    upair = _reformat_tc(user_table.T, NUSERS)
    mpair = _reformat_tc(movie_table.T, NMOVIES)
    u2, m2 = _gather_sc(user_ids, movie_ids, upair, mpair)
    return _mlp_tc(u2, m2, user_ids, movie_ids, movie_features,
                   W1, b1, W2, b2, W3, b3)


# speedup vs baseline: 1.1105x; 1.1105x over previous
import functools

import jax
import jax.numpy as jnp
from jax import lax
from jax.experimental import pallas as pl
from jax.experimental.pallas import tpu as pltpu
from jax.experimental.pallas import tpu_sc as plsc

B = 16384
D = 64
F = 16
H1 = 128
H2 = 64
NUSERS = 1000000
NMOVIES = 100000

NC = 2
NS = 16
NW = NC * NS
BPW = B // NW
CHUNK = 128
L = 16

_sc_mesh = plsc.VectorSubcoreMesh(core_axis_name="c", subcore_axis_name="s")

CB = 1024


def _reformat_body(tT_ref, o_ref):
    x = tT_ref[...]
    o_ref[:, 0:D] = x[:, 0:CB // 2].T
    o_ref[:, D:2 * D] = x[:, CB // 2:CB].T


def _reformat_tc(tT, n):
    grid = (pl.cdiv(n, CB),)
    return pl.pallas_call(
        _reformat_body,
        grid=grid,
        in_specs=[pl.BlockSpec((D, CB), lambda j: (0, j))],
        out_specs=pl.BlockSpec((CB // 2, 2 * D), lambda j: (j, 0)),
        out_shape=jax.ShapeDtypeStruct((n // 2, 2 * D), jnp.float32),
    )(tT)


def _gather_one(ids_hbm, pair_hbm, out_hbm, base, idx_v, tidx_v, rows_v, sem):
    pltpu.sync_copy(ids_hbm.at[pl.ds(base, BPW)], idx_v)

    def _shift(i, carry):
        sl = pl.ds(i * L, L)
        v = idx_v[sl]
        tidx_v[sl] = lax.shift_left(lax.shift_right_logical(v, 10), 9) | (v & 511)
        return carry

    lax.fori_loop(0, BPW // L, _shift, 0, unroll=4)

    copies = []
    for j in range(BPW // CHUNK):
        sl = pl.ds(j * CHUNK, CHUNK)
        copies.append(
            pltpu.async_copy(pair_hbm.at[tidx_v.at[sl]], rows_v.at[sl], sem))
    for cp in copies:
        cp.wait()
    pltpu.sync_copy(rows_v, out_hbm.at[pl.ds(base, BPW)])


@functools.partial(
    pl.kernel,
    mesh=_sc_mesh,
    out_type=[
        jax.ShapeDtypeStruct((B, 2 * D), jnp.float32),
        jax.ShapeDtypeStruct((B, 2 * D), jnp.float32),
    ],
    scratch_types=[
        pltpu.VMEM((BPW,), jnp.int32),
        pltpu.VMEM((BPW,), jnp.int32),
        pltpu.VMEM((BPW, 2 * D), jnp.float32),
        pltpu.SemaphoreType.DMA,
    ],
)
def _gather_sc(uids_hbm, mids_hbm, upair_hbm, mpair_hbm, uout_hbm, mout_hbm,
               idx_v, tidx_v, rows_v, sem):
    wid = lax.axis_index("s") * NC + lax.axis_index("c")
    base = wid * BPW
    _gather_one(uids_hbm, upair_hbm, uout_hbm, base, idx_v, tidx_v, rows_v,
                sem)
    _gather_one(mids_hbm, mpair_hbm, mout_hbm, base, idx_v, tidx_v, rows_v,
                sem)


BB = 512


def _mlp_body(u2_ref, m2_ref, uid_ref, mid_ref, f_ref, w1_ref, b1_ref,
              w2_ref, b2_ref, w3t_ref, b3_ref, o_ref):
    u2 = u2_ref[...]
    m2 = m2_ref[...]
    up = ((uid_ref[...] >> 9) & 1)[:, None] == 1
    mp = ((mid_ref[...] >> 9) & 1)[:, None] == 1
    u = jnp.where(up, u2[:, D:2 * D], u2[:, 0:D])
    m = jnp.where(mp, m2[:, D:2 * D], m2[:, 0:D])
    w1 = w1_ref[...]
    h1 = (
        jnp.dot(u, w1[0:D, :], preferred_element_type=jnp.float32)
        + jnp.dot(m, w1[D:2 * D, :], preferred_element_type=jnp.float32)
        + jnp.dot(f_ref[...], w1[2 * D:2 * D + F, :],
                  preferred_element_type=jnp.float32)
        + b1_ref[...]
    )
    h1 = jnp.maximum(h1, 0.0)
    h2 = jnp.maximum(
        jnp.dot(h1, w2_ref[...], preferred_element_type=jnp.float32)
        + b2_ref[...], 0.0)
    z = jnp.sum(h2 * w3t_ref[...], axis=1, keepdims=True) + b3_ref[...]
    o_ref[...] = jax.nn.sigmoid(z)


@jax.jit
def _mlp_tc(u2, m2, uids, mids, f, W1, b1, W2, b2, W3, b3):
    grid = (B // BB,)
    return pl.pallas_call(
        _mlp_body,
        grid=grid,
        in_specs=[
            pl.BlockSpec((BB, 2 * D), lambda i: (i, 0)),
            pl.BlockSpec((BB, 2 * D), lambda i: (i, 0)),
            pl.BlockSpec((BB,), lambda i: (i,)),
            pl.BlockSpec((BB,), lambda i: (i,)),
            pl.BlockSpec((BB, F), lambda i: (i, 0)),
            pl.BlockSpec((2 * D + F, H1), lambda i: (0, 0)),
            pl.BlockSpec((1, H1), lambda i: (0, 0)),
            pl.BlockSpec((H1, H2), lambda i: (0, 0)),
            pl.BlockSpec((1, H2), lambda i: (0, 0)),
            pl.BlockSpec((1, H2), lambda i: (0, 0)),
            pl.BlockSpec((1, 1), lambda i: (0, 0)),
        ],
        out_specs=pl.BlockSpec((BB, 1), lambda i: (i, 0)),
        out_shape=jax.ShapeDtypeStruct((B, 1), jnp.float32),
    )(u2, m2, uids, mids, f, W1, b1.reshape(1, H1), W2, b2.reshape(1, H2),
      W3.reshape(1, H2), b3.reshape(1, 1))


def kernel(user_ids, movie_ids, movie_features, user_table, movie_table,
           W1, b1, W2, b2, W3, b3):
    upair = _reformat_tc(user_table.T, NUSERS)
    mpair = _reformat_tc(movie_table.T, NMOVIES)
    u2, m2 = _gather_sc(user_ids, movie_ids, upair, mpair)
    return _mlp_tc(u2, m2, user_ids, movie_ids, movie_features,
                   W1, b1, W2, b2, W3, b3)


# speedup vs baseline: 2.1472x; 1.9335x over previous
import functools

import jax
import jax.numpy as jnp
from jax import lax
from jax.experimental import pallas as pl
from jax.experimental.pallas import tpu as pltpu
from jax.experimental.pallas import tpu_sc as plsc

B = 16384
D = 64
F = 16
H1 = 128
H2 = 64
NUSERS = 1000000
NMOVIES = 100000

NC = 2
NS = 16
NW = NC * NS
BPW = B // NW
CHUNK = 128
L = 16

_sc_mesh = plsc.VectorSubcoreMesh(core_axis_name="c", subcore_axis_name="s")

CB = 4096


def _reformat_body(tT_ref, o_ref):
    x = tT_ref[...]
    o_ref[:, 0:D] = x[:, 0:CB // 2].T
    o_ref[:, D:2 * D] = x[:, CB // 2:CB].T


def _reformat_tc(tT, n):
    grid = (pl.cdiv(n, CB),)
    return pl.pallas_call(
        _reformat_body,
        grid=grid,
        in_specs=[pl.BlockSpec((D, CB), lambda j: (0, j))],
        out_specs=pl.BlockSpec((CB // 2, 2 * D), lambda j: (j, 0)),
        out_shape=jax.ShapeDtypeStruct((n // 2, 2 * D), jnp.float32),
    )(tT)


def _gather_one(ids_hbm, pair_hbm, out_hbm, base, idx_v, tidx_v, rows_v, sem):
    pltpu.sync_copy(ids_hbm.at[pl.ds(base, BPW)], idx_v)

    def _shift(i, carry):
        sl = pl.ds(i * L, L)
        v = idx_v[sl]
        tidx_v[sl] = lax.shift_left(lax.shift_right_logical(v, 12), 11) | (v & 2047)
        return carry

    lax.fori_loop(0, BPW // L, _shift, 0, unroll=4)

    copies = []
    for j in range(BPW // CHUNK):
        sl = pl.ds(j * CHUNK, CHUNK)
        copies.append(
            pltpu.async_copy(pair_hbm.at[tidx_v.at[sl]], rows_v.at[sl], sem))
    for cp in copies:
        cp.wait()
    pltpu.sync_copy(rows_v, out_hbm.at[pl.ds(base, BPW)])


@functools.partial(
    pl.kernel,
    mesh=_sc_mesh,
    out_type=[
        jax.ShapeDtypeStruct((B, 2 * D), jnp.float32),
        jax.ShapeDtypeStruct((B, 2 * D), jnp.float32),
    ],
    scratch_types=[
        pltpu.VMEM((BPW,), jnp.int32),
        pltpu.VMEM((BPW,), jnp.int32),
        pltpu.VMEM((BPW, 2 * D), jnp.float32),
        pltpu.SemaphoreType.DMA,
    ],
)
def _gather_sc(uids_hbm, mids_hbm, upair_hbm, mpair_hbm, uout_hbm, mout_hbm,
               idx_v, tidx_v, rows_v, sem):
    wid = lax.axis_index("s") * NC + lax.axis_index("c")
    base = wid * BPW
    _gather_one(uids_hbm, upair_hbm, uout_hbm, base, idx_v, tidx_v, rows_v,
                sem)
    _gather_one(mids_hbm, mpair_hbm, mout_hbm, base, idx_v, tidx_v, rows_v,
                sem)


BB = 512


def _mlp_body(u2_ref, m2_ref, uid_ref, mid_ref, f_ref, w1_ref, b1_ref,
              w2_ref, b2_ref, w3t_ref, b3_ref, o_ref):
    u2 = u2_ref[...]
    m2 = m2_ref[...]
    up = ((uid_ref[...] >> 11) & 1)[:, None] == 1
    mp = ((mid_ref[...] >> 11) & 1)[:, None] == 1
    u = jnp.where(up, u2[:, D:2 * D], u2[:, 0:D])
    m = jnp.where(mp, m2[:, D:2 * D], m2[:, 0:D])
    w1 = w1_ref[...]
    h1 = (
        jnp.dot(u, w1[0:D, :], preferred_element_type=jnp.float32)
        + jnp.dot(m, w1[D:2 * D, :], preferred_element_type=jnp.float32)
        + jnp.dot(f_ref[...], w1[2 * D:2 * D + F, :],
                  preferred_element_type=jnp.float32)
        + b1_ref[...]
    )
    h1 = jnp.maximum(h1, 0.0)
    h2 = jnp.maximum(
        jnp.dot(h1, w2_ref[...], preferred_element_type=jnp.float32)
        + b2_ref[...], 0.0)
    z = jnp.sum(h2 * w3t_ref[...], axis=1, keepdims=True) + b3_ref[...]
    o_ref[...] = jax.nn.sigmoid(z)


@jax.jit
def _mlp_tc(u2, m2, uids, mids, f, W1, b1, W2, b2, W3, b3):
    grid = (B // BB,)
    return pl.pallas_call(
        _mlp_body,
        grid=grid,
        in_specs=[
            pl.BlockSpec((BB, 2 * D), lambda i: (i, 0)),
            pl.BlockSpec((BB, 2 * D), lambda i: (i, 0)),
            pl.BlockSpec((BB,), lambda i: (i,)),
            pl.BlockSpec((BB,), lambda i: (i,)),
            pl.BlockSpec((BB, F), lambda i: (i, 0)),
            pl.BlockSpec((2 * D + F, H1), lambda i: (0, 0)),
            pl.BlockSpec((1, H1), lambda i: (0, 0)),
            pl.BlockSpec((H1, H2), lambda i: (0, 0)),
            pl.BlockSpec((1, H2), lambda i: (0, 0)),
            pl.BlockSpec((1, H2), lambda i: (0, 0)),
            pl.BlockSpec((1, 1), lambda i: (0, 0)),
        ],
        out_specs=pl.BlockSpec((BB, 1), lambda i: (i, 0)),
        out_shape=jax.ShapeDtypeStruct((B, 1), jnp.float32),
    )(u2, m2, uids, mids, f, W1, b1.reshape(1, H1), W2, b2.reshape(1, H2),
      W3.reshape(1, H2), b3.reshape(1, 1))


def kernel(user_ids, movie_ids, movie_features, user_table, movie_table,
           W1, b1, W2, b2, W3, b3):
    upair = _reformat_tc(user_table.T, NUSERS)
    mpair = _reformat_tc(movie_table.T, NMOVIES)
    u2, m2 = _gather_sc(user_ids, movie_ids, upair, mpair)
    return _mlp_tc(u2, m2, user_ids, movie_ids, movie_features,
                   W1, b1, W2, b2, W3, b3)


# speedup vs baseline: 2.8673x; 1.3354x over previous
import functools

import jax
import jax.numpy as jnp
from jax import lax
from jax.experimental import pallas as pl
from jax.experimental.pallas import tpu as pltpu
from jax.experimental.pallas import tpu_sc as plsc

B = 16384
D = 64
F = 16
H1 = 128
H2 = 64
NUSERS = 1000000
NMOVIES = 100000

NC = 2
NS = 16
NW = NC * NS
BPW = B // NW
CHUNK = 128
L = 16

_sc_mesh = plsc.VectorSubcoreMesh(core_axis_name="c", subcore_axis_name="s")

CB = 16384


def _reformat_body(tT_ref, o_ref):
    x = tT_ref[...]
    o_ref[:, 0:D] = x[:, 0:CB // 2].T
    o_ref[:, D:2 * D] = x[:, CB // 2:CB].T


def _reformat_tc(tT, n):
    grid = (pl.cdiv(n, CB),)
    return pl.pallas_call(
        _reformat_body,
        grid=grid,
        in_specs=[pl.BlockSpec((D, CB), lambda j: (0, j))],
        out_specs=pl.BlockSpec((CB // 2, 2 * D), lambda j: (j, 0)),
        out_shape=jax.ShapeDtypeStruct((n // 2, 2 * D), jnp.float32),
    )(tT)


def _gather_one(ids_hbm, pair_hbm, out_hbm, base, idx_v, tidx_v, rows_v, sem):
    pltpu.sync_copy(ids_hbm.at[pl.ds(base, BPW)], idx_v)

    def _shift(i, carry):
        sl = pl.ds(i * L, L)
        v = idx_v[sl]
        tidx_v[sl] = lax.shift_left(lax.shift_right_logical(v, 14), 13) | (v & 8191)
        return carry

    lax.fori_loop(0, BPW // L, _shift, 0, unroll=4)

    copies = []
    for j in range(BPW // CHUNK):
        sl = pl.ds(j * CHUNK, CHUNK)
        copies.append(
            pltpu.async_copy(pair_hbm.at[tidx_v.at[sl]], rows_v.at[sl], sem))
    for cp in copies:
        cp.wait()
    pltpu.sync_copy(rows_v, out_hbm.at[pl.ds(base, BPW)])


@functools.partial(
    pl.kernel,
    mesh=_sc_mesh,
    out_type=[
        jax.ShapeDtypeStruct((B, 2 * D), jnp.float32),
        jax.ShapeDtypeStruct((B, 2 * D), jnp.float32),
    ],
    scratch_types=[
        pltpu.VMEM((BPW,), jnp.int32),
        pltpu.VMEM((BPW,), jnp.int32),
        pltpu.VMEM((BPW, 2 * D), jnp.float32),
        pltpu.SemaphoreType.DMA,
    ],
)
def _gather_sc(uids_hbm, mids_hbm, upair_hbm, mpair_hbm, uout_hbm, mout_hbm,
               idx_v, tidx_v, rows_v, sem):
    wid = lax.axis_index("s") * NC + lax.axis_index("c")
    base = wid * BPW
    _gather_one(uids_hbm, upair_hbm, uout_hbm, base, idx_v, tidx_v, rows_v,
                sem)
    _gather_one(mids_hbm, mpair_hbm, mout_hbm, base, idx_v, tidx_v, rows_v,
                sem)


BB = 512


def _mlp_body(u2_ref, m2_ref, uid_ref, mid_ref, f_ref, w1_ref, b1_ref,
              w2_ref, b2_ref, w3t_ref, b3_ref, o_ref):
    u2 = u2_ref[...]
    m2 = m2_ref[...]
    up = ((uid_ref[...] >> 13) & 1)[:, None] == 1
    mp = ((mid_ref[...] >> 13) & 1)[:, None] == 1
    u = jnp.where(up, u2[:, D:2 * D], u2[:, 0:D])
    m = jnp.where(mp, m2[:, D:2 * D], m2[:, 0:D])
    w1 = w1_ref[...]
    h1 = (
        jnp.dot(u, w1[0:D, :], preferred_element_type=jnp.float32)
        + jnp.dot(m, w1[D:2 * D, :], preferred_element_type=jnp.float32)
        + jnp.dot(f_ref[...], w1[2 * D:2 * D + F, :],
                  preferred_element_type=jnp.float32)
        + b1_ref[...]
    )
    h1 = jnp.maximum(h1, 0.0)
    h2 = jnp.maximum(
        jnp.dot(h1, w2_ref[...], preferred_element_type=jnp.float32)
        + b2_ref[...], 0.0)
    z = jnp.sum(h2 * w3t_ref[...], axis=1, keepdims=True) + b3_ref[...]
    o_ref[...] = jax.nn.sigmoid(z)


@jax.jit
def _mlp_tc(u2, m2, uids, mids, f, W1, b1, W2, b2, W3, b3):
    grid = (B // BB,)
    return pl.pallas_call(
        _mlp_body,
        grid=grid,
        in_specs=[
            pl.BlockSpec((BB, 2 * D), lambda i: (i, 0)),
            pl.BlockSpec((BB, 2 * D), lambda i: (i, 0)),
            pl.BlockSpec((BB,), lambda i: (i,)),
            pl.BlockSpec((BB,), lambda i: (i,)),
            pl.BlockSpec((BB, F), lambda i: (i, 0)),
            pl.BlockSpec((2 * D + F, H1), lambda i: (0, 0)),
            pl.BlockSpec((1, H1), lambda i: (0, 0)),
            pl.BlockSpec((H1, H2), lambda i: (0, 0)),
            pl.BlockSpec((1, H2), lambda i: (0, 0)),
            pl.BlockSpec((1, H2), lambda i: (0, 0)),
            pl.BlockSpec((1, 1), lambda i: (0, 0)),
        ],
        out_specs=pl.BlockSpec((BB, 1), lambda i: (i, 0)),
        out_shape=jax.ShapeDtypeStruct((B, 1), jnp.float32),
    )(u2, m2, uids, mids, f, W1, b1.reshape(1, H1), W2, b2.reshape(1, H2),
      W3.reshape(1, H2), b3.reshape(1, 1))


def kernel(user_ids, movie_ids, movie_features, user_table, movie_table,
           W1, b1, W2, b2, W3, b3):
    upair = _reformat_tc(user_table.T, NUSERS)
    mpair = _reformat_tc(movie_table.T, NMOVIES)
    u2, m2 = _gather_sc(user_ids, movie_ids, upair, mpair)
    return _mlp_tc(u2, m2, user_ids, movie_ids, movie_features,
                   W1, b1, W2, b2, W3, b3)


# speedup vs baseline: 2.9634x; 1.0335x over previous
import functools

import jax
import jax.numpy as jnp
from jax import lax
from jax.experimental import pallas as pl
from jax.experimental.pallas import tpu as pltpu
from jax.experimental.pallas import tpu_sc as plsc

B = 16384
D = 64
F = 16
H1 = 128
H2 = 64
NUSERS = 1000000
NMOVIES = 100000

NC = 2
NS = 16
NW = NC * NS
BPW = B // NW
CHUNK = 128
L = 16

_sc_mesh = plsc.VectorSubcoreMesh(core_axis_name="c", subcore_axis_name="s")

CB = 32768


def _reformat_body(tT_ref, o_ref):
    x = tT_ref[...]
    o_ref[:, 0:D] = x[:, 0:CB // 2].T
    o_ref[:, D:2 * D] = x[:, CB // 2:CB].T


def _reformat_tc(tT, n):
    grid = (pl.cdiv(n, CB),)
    return pl.pallas_call(
        _reformat_body,
        grid=grid,
        in_specs=[pl.BlockSpec((D, CB), lambda j: (0, j))],
        out_specs=pl.BlockSpec((CB // 2, 2 * D), lambda j: (j, 0)),
        out_shape=jax.ShapeDtypeStruct((n // 2, 2 * D), jnp.float32),
    )(tT)


def _gather_one(ids_hbm, pair_hbm, out_hbm, base, idx_v, tidx_v, rows_v, sem):
    pltpu.sync_copy(ids_hbm.at[pl.ds(base, BPW)], idx_v)

    def _shift(i, carry):
        sl = pl.ds(i * L, L)
        v = idx_v[sl]
        tidx_v[sl] = lax.shift_left(lax.shift_right_logical(v, 15), 14) | (v & 16383)
        return carry

    lax.fori_loop(0, BPW // L, _shift, 0, unroll=4)

    copies = []
    for j in range(BPW // CHUNK):
        sl = pl.ds(j * CHUNK, CHUNK)
        copies.append(
            pltpu.async_copy(pair_hbm.at[tidx_v.at[sl]], rows_v.at[sl], sem))
    for cp in copies:
        cp.wait()
    pltpu.sync_copy(rows_v, out_hbm.at[pl.ds(base, BPW)])


@functools.partial(
    pl.kernel,
    mesh=_sc_mesh,
    out_type=[
        jax.ShapeDtypeStruct((B, 2 * D), jnp.float32),
        jax.ShapeDtypeStruct((B, 2 * D), jnp.float32),
    ],
    scratch_types=[
        pltpu.VMEM((BPW,), jnp.int32),
        pltpu.VMEM((BPW,), jnp.int32),
        pltpu.VMEM((BPW, 2 * D), jnp.float32),
        pltpu.SemaphoreType.DMA,
    ],
)
def _gather_sc(uids_hbm, mids_hbm, upair_hbm, mpair_hbm, uout_hbm, mout_hbm,
               idx_v, tidx_v, rows_v, sem):
    wid = lax.axis_index("s") * NC + lax.axis_index("c")
    base = wid * BPW
    _gather_one(uids_hbm, upair_hbm, uout_hbm, base, idx_v, tidx_v, rows_v,
                sem)
    _gather_one(mids_hbm, mpair_hbm, mout_hbm, base, idx_v, tidx_v, rows_v,
                sem)


BB = 512


def _mlp_body(u2_ref, m2_ref, uid_ref, mid_ref, f_ref, w1_ref, b1_ref,
              w2_ref, b2_ref, w3t_ref, b3_ref, o_ref):
    u2 = u2_ref[...]
    m2 = m2_ref[...]
    up = ((uid_ref[...] >> 14) & 1)[:, None] == 1
    mp = ((mid_ref[...] >> 14) & 1)[:, None] == 1
    u = jnp.where(up, u2[:, D:2 * D], u2[:, 0:D])
    m = jnp.where(mp, m2[:, D:2 * D], m2[:, 0:D])
    w1 = w1_ref[...]
    h1 = (
        jnp.dot(u, w1[0:D, :], preferred_element_type=jnp.float32)
        + jnp.dot(m, w1[D:2 * D, :], preferred_element_type=jnp.float32)
        + jnp.dot(f_ref[...], w1[2 * D:2 * D + F, :],
                  preferred_element_type=jnp.float32)
        + b1_ref[...]
    )
    h1 = jnp.maximum(h1, 0.0)
    h2 = jnp.maximum(
        jnp.dot(h1, w2_ref[...], preferred_element_type=jnp.float32)
        + b2_ref[...], 0.0)
    z = jnp.sum(h2 * w3t_ref[...], axis=1, keepdims=True) + b3_ref[...]
    o_ref[...] = jax.nn.sigmoid(z)


@jax.jit
def _mlp_tc(u2, m2, uids, mids, f, W1, b1, W2, b2, W3, b3):
    grid = (B // BB,)
    return pl.pallas_call(
        _mlp_body,
        grid=grid,
        in_specs=[
            pl.BlockSpec((BB, 2 * D), lambda i: (i, 0)),
            pl.BlockSpec((BB, 2 * D), lambda i: (i, 0)),
            pl.BlockSpec((BB,), lambda i: (i,)),
            pl.BlockSpec((BB,), lambda i: (i,)),
            pl.BlockSpec((BB, F), lambda i: (i, 0)),
            pl.BlockSpec((2 * D + F, H1), lambda i: (0, 0)),
            pl.BlockSpec((1, H1), lambda i: (0, 0)),
            pl.BlockSpec((H1, H2), lambda i: (0, 0)),
            pl.BlockSpec((1, H2), lambda i: (0, 0)),
            pl.BlockSpec((1, H2), lambda i: (0, 0)),
            pl.BlockSpec((1, 1), lambda i: (0, 0)),
        ],
        out_specs=pl.BlockSpec((BB, 1), lambda i: (i, 0)),
        out_shape=jax.ShapeDtypeStruct((B, 1), jnp.float32),
    )(u2, m2, uids, mids, f, W1, b1.reshape(1, H1), W2, b2.reshape(1, H2),
      W3.reshape(1, H2), b3.reshape(1, 1))


def kernel(user_ids, movie_ids, movie_features, user_table, movie_table,
           W1, b1, W2, b2, W3, b3):
    upair = _reformat_tc(user_table.T, NUSERS)
    mpair = _reformat_tc(movie_table.T, NMOVIES)
    u2, m2 = _gather_sc(user_ids, movie_ids, upair, mpair)
    return _mlp_tc(u2, m2, user_ids, movie_ids, movie_features,
                   W1, b1, W2, b2, W3, b3)
